# baseline (speedup 1.0000x reference)
import functools

import jax
import jax.numpy as jnp
from jax import lax
from jax.experimental import pallas as pl
from jax.experimental.pallas import tpu as pltpu
from jax.experimental.pallas import tpu_sc as plsc

_D = 1024
_L = 16


@functools.lru_cache(maxsize=None)
def _make_sc_kernel(N, CH, NC, NS):
    NW = NC * NS
    rows_per_w = N // NW
    n_chunks = rows_per_w // CH
    mesh = plsc.VectorSubcoreMesh(core_axis_name="c", subcore_axis_name="s")

    @functools.partial(
        pl.kernel,
        mesh=mesh,
        out_type=jax.ShapeDtypeStruct((N, _D), jnp.float32),
        scratch_types=[
            pltpu.VMEM((CH,), jnp.int32),
            pltpu.VMEM((CH, _D), jnp.float32),
            pltpu.VMEM((CH, _D), jnp.float32),
            pltpu.SemaphoreType.DMA,
        ],
    )
    def k(x_hbm, mask_hbm, w_hbm, out_hbm, idx_v, xbuf, ebuf, sem):
        c = lax.axis_index("c")
        s = lax.axis_index("s")
        wid = s * NC + c
        base = wid * rows_per_w

        def chunk_body(i, carry):
            row0 = base + i * CH
            pltpu.sync_copy(mask_hbm.at[pl.ds(row0, CH)], idx_v)
            pltpu.sync_copy(x_hbm.at[pl.ds(row0, CH)], xbuf)
            pltpu.async_copy(w_hbm.at[idx_v], ebuf, sem).wait()

            def add_row(r, carry2):
                def add_vec(j, carry3):
                    sl = pl.ds(j * _L, _L)
                    xbuf[r, sl] = xbuf[r, sl] + ebuf[r, sl]
                    return carry3

                return lax.fori_loop(0, _D // _L, add_vec, carry2)

            lax.fori_loop(0, CH, add_row, 0)
            pltpu.sync_copy(xbuf, out_hbm.at[pl.ds(row0, CH)])
            return carry

        lax.fori_loop(0, n_chunks, chunk_body, 0)

    return k


def kernel(x, motion_mask, motion_embed_weight):
    B, P, D = x.shape
    if motion_mask.ndim == 1:
        motion_mask = jnp.broadcast_to(motion_mask[None, :], (B, P))
    mask = motion_mask.astype(jnp.int32).reshape(-1)
    N = B * P
    info = plsc.get_sparse_core_info()
    k = _make_sc_kernel(N, 32, info.num_cores, info.num_subcores)
    out = k(x.reshape(N, D), mask, motion_embed_weight)
    return out.reshape(B, P, D)


# speedup vs baseline: 2.2746x; 2.2746x over previous
import functools

import jax
import jax.numpy as jnp
from jax import lax
from jax.experimental import pallas as pl
from jax.experimental.pallas import tpu as pltpu
from jax.experimental.pallas import tpu_sc as plsc

_D = 1024
_L = 16


@functools.lru_cache(maxsize=None)
def _make_sc_kernel(N, CH, NC, NS):
    NW = NC * NS
    rows_per_w = N // NW
    n_chunks = rows_per_w // CH
    assert n_chunks % 2 == 0
    mesh = plsc.VectorSubcoreMesh(core_axis_name="c", subcore_axis_name="s")

    @functools.partial(
        pl.kernel,
        mesh=mesh,
        out_type=jax.ShapeDtypeStruct((N, _D), jnp.float32),
        compiler_params=pltpu.CompilerParams(needs_layout_passes=False),
        scratch_types=[
            pltpu.VMEM((CH,), jnp.int32),
            pltpu.VMEM((CH,), jnp.int32),
            pltpu.VMEM((2, CH, _D), jnp.float32),
            pltpu.VMEM((3 * _D,), jnp.float32),
            pltpu.SemaphoreType.DMA,
            pltpu.SemaphoreType.DMA,
            pltpu.SemaphoreType.DMA,
            pltpu.SemaphoreType.DMA,
        ],
    )
    def k(x_hbm, mask_hbm, w_hbm, out_hbm, idx0, idx1, xbuf, w_v, sx0, sx1, so0, so1):
        idx_refs = (idx0, idx1)
        c = lax.axis_index("c")
        s = lax.axis_index("s")
        wid = s * NC + c
        base = wid * rows_per_w
        sx = (sx0, sx1)
        so = (so0, so1)
        iota = lax.iota(jnp.int32, _L)

        pltpu.sync_copy(w_hbm, w_v)

        def issue_in(i, b):
            row0 = base + i * CH
            pltpu.sync_copy(mask_hbm.at[pl.ds(row0, CH)], idx_refs[b])
            pltpu.async_copy(x_hbm.at[pl.ds(row0, CH)], xbuf.at[b], sx[b])

        def wait_in(b):
            pltpu.make_async_copy(x_hbm.at[pl.ds(0, CH)], xbuf.at[b], sx[b]).wait()

        def wait_out(b):
            pltpu.make_async_copy(xbuf.at[b], out_hbm.at[pl.ds(0, CH)], so[b]).wait()

        def compute_and_flush(i, b):
            wait_in(b)

            def add_row(r, carry):
                rb = jnp.full((_L,), r, jnp.int32)
                m = plsc.load_gather(idx_refs[b], [rb])
                mi = m * _D + iota
                for j in range(_D // _L):
                    v = plsc.load_gather(w_v, [mi + (j * _L)])
                    plsc.addupdate(xbuf.at[b, r, pl.ds(j * _L, _L)], v)
                return carry

            lax.fori_loop(0, CH, add_row, 0)
            pltpu.async_copy(xbuf.at[b], out_hbm.at[pl.ds(base + i * CH, CH)], so[b])

        issue_in(0, 0)

        def pair_body(g, carry):
            for b in range(2):
                i = 2 * g + b
                nb = (b + 1) % 2

                @pl.when(i + 1 < n_chunks)
                def _():
                    @pl.when(i >= 1)
                    def _():
                        wait_out(nb)

                    issue_in(i + 1, nb)

                compute_and_flush(i, b)
            return carry

        lax.fori_loop(0, n_chunks // 2, pair_body, 0)
        wait_out(0)
        wait_out(1)

    return k


def kernel(x, motion_mask, motion_embed_weight):
    B, P, D = x.shape
    if motion_mask.ndim == 1:
        motion_mask = jnp.broadcast_to(motion_mask[None, :], (B, P))
    mask = motion_mask.astype(jnp.int32).reshape(-1)
    N = B * P
    info = plsc.get_sparse_core_info()
    k = _make_sc_kernel(N, 32, info.num_cores, info.num_subcores)
    out = k(x.reshape(N, D), mask, motion_embed_weight.reshape(-1))
    return out.reshape(B, P, D)
